# baseline (speedup 1.0000x reference)
import functools

import jax
import jax.numpy as jnp
from jax import lax
from jax.experimental import pallas as pl
from jax.experimental.pallas import tpu as pltpu
from jax.experimental.pallas import tpu_sc as plsc

_LANES = 16
_CHUNK = 1024
_GSUB = 128


_CH = 512
_PP = _CH // _GSUB


@functools.partial(jax.jit, static_argnums=(3, 4, 5, 6, 7, 8))
def _masked_gather(weight, mask, widx, V, D, N, NW, B, H):
    PIECES = N // _GSUB
    PPW = PIECES // NW
    n_chunks = PPW // _PP
    BB = B // 128
    DBLK = D // 8
    info = plsc.get_sparse_core_info()
    nc = info.num_cores

    mesh = plsc.VectorSubcoreMesh(core_axis_name="c", subcore_axis_name="s")

    @functools.partial(
        pl.kernel,
        mesh=mesh,
        out_type=jax.ShapeDtypeStruct((N * D,), jnp.float32),
        scratch_types=[
            pltpu.VMEM((2, _CH), jnp.int32),
            pltpu.VMEM((2, _CH), jnp.float32),
            pltpu.VMEM((2, _CH, D), jnp.float32),
            pltpu.VMEM((2, _PP, DBLK * 8 * _GSUB), jnp.float32),
            pltpu.SemaphoreType.DMA,
            pltpu.SemaphoreType.DMA,
            pltpu.SemaphoreType.DMA,
            pltpu.SemaphoreType.DMA,
        ],
        compiler_params=pltpu.CompilerParams(
            use_tc_tiling_on_sc=False, needs_layout_passes=False,
            disable_bounds_checks=True),
    )
    def gather_kernel(weight_hbm, mask_hbm, widx_hbm, out_hbm,
                      idx_v, mval_v, rows_v, obuf,
                      gsem0, gsem1, osem0, osem1):
        wid = lax.axis_index("s") * nc + lax.axis_index("c")
        gsem = (gsem0, gsem1)
        osem = (osem0, osem1)

        def fire_gathers(c, s):
            start = pl.multiple_of((wid * PPW + c * _PP) * _GSUB, _CH)
            pltpu.sync_copy(widx_hbm.at[pl.ds(start, _CH)], idx_v.at[s])
            for j in range(_PP):
                sub = idx_v.at[s, pl.ds(j * _GSUB, _GSUB)]
                pltpu.async_copy(
                    weight_hbm.at[sub],
                    rows_v.at[s, pl.ds(j * _GSUB, _GSUB)], gsem[s])
                pltpu.async_copy(
                    mask_hbm.at[sub],
                    mval_v.at[s, pl.ds(j * _GSUB, _GSUB)], gsem[s])

        def drain_gathers(s):
            pltpu.make_async_copy(
                weight_hbm.at[pl.ds(0, _CH), :], rows_v.at[s], gsem[s]).wait()
            pltpu.make_async_copy(
                mask_hbm.at[pl.ds(0, _CH)], mval_v.at[s], gsem[s]).wait()

        def fire_outs(c, s):
            gp0 = wid * PPW + c * _PP
            for p in range(_PP):
                gp = gp0 + p
                hblk = gp // (BB * 8)
                rem = gp % (BB * 8)
                bblk = rem // 8
                hsub = rem % 8
                h = hblk * 8 + hsub
                base = h * (D * B) + bblk * 1024
                for dblk in range(DBLK):
                    off = pl.multiple_of(base + dblk * (8 * B), 1024)
                    pltpu.async_copy(
                        obuf.at[s, p, pl.ds(dblk * 1024, 1024)],
                        out_hbm.at[pl.ds(off, 1024)], osem[s])

        def drain_outs(s):
            for p in range(_PP):
                pltpu.make_async_copy(
                    obuf.at[s, p], out_hbm.at[pl.ds(0, DBLK * 8 * _GSUB)],
                    osem[s]).wait()

        def compute(s):
            def piece_body(p, carry2):
                lanes = lax.iota(jnp.int32, _LANES)
                for dg in range(D // 8):
                    rots = [(dg * 8 + dd + lanes) & (D - 1) for dd in range(8)]
                    opos = [((r >> 3) << 10) + ((r & 7) << 7) for r in rots]
                    for jb in range(8):
                        rbase = p * _GSUB + jb * _LANES
                        rowvec = rbase + lanes
                        mvec = mval_v[s, pl.ds(rbase, _LANES)]
                        obase = jb * _LANES + lanes
                        vals = [plsc.load_gather(rows_v.at[s], [rowvec, rots[dd]])
                                for dd in range(8)]
                        for dd in range(8):
                            plsc.store_scatter(
                                obuf.at[s, p], [opos[dd] + obase],
                                vals[dd] * mvec)
                return carry2

            lax.fori_loop(0, _PP, piece_body, 0)

        fire_gathers(0, 0)

        def pair_body(cp, carry):
            for s in range(2):
                c = cp * 2 + s

                @pl.when(c + 1 < n_chunks)
                def _():
                    fire_gathers(c + 1, 1 - s)

                drain_gathers(s)

                @pl.when(c >= 2)
                def _():
                    drain_outs(s)

                compute(s)
                fire_outs(c, s)
            return carry

        lax.fori_loop(0, n_chunks // 2, pair_body, 0)
        drain_outs(0)
        drain_outs(1)

    return gather_kernel(weight, mask, widx)


def kernel(weight, words, p):
    V, D = weight.shape
    B, H = words.shape
    N = B * H
    mask = jax.random.bernoulli(
        jax.random.key(42), p, (V,)).astype(jnp.float32) / p
    info = plsc.get_sparse_core_info()
    NW = info.num_cores * info.num_subcores
    widx = (words.astype(jnp.int32)
            .reshape(B // 128, 128, H // 8, 8)
            .transpose(2, 0, 3, 1)
            .reshape(N))
    out1d = _masked_gather(weight, mask, widx, V, D, N, NW, B, H)
    a = out1d.reshape(H, D // 8, B // 128, 8, 128)
    return a.transpose(2, 4, 0, 1, 3).reshape(B, H, D)


# speedup vs baseline: 1.3402x; 1.3402x over previous
import functools

import jax
import jax.numpy as jnp
from jax import lax
from jax.experimental import pallas as pl
from jax.experimental.pallas import tpu as pltpu
from jax.experimental.pallas import tpu_sc as plsc

_LANES = 16
_CHUNK = 1024
_GSUB = 128


_CH = 512
_PP = _CH // _GSUB


@functools.partial(jax.jit, static_argnums=(1, 2))
def _detile_weight(wT, V, D):
    VBF = V // 128
    SB = VBF // 4
    n_iter = (SB + 31) // 32

    info = plsc.get_sparse_core_info()
    nc = info.num_cores
    mesh = plsc.VectorSubcoreMesh(core_axis_name="c", subcore_axis_name="s")

    @functools.partial(
        pl.kernel,
        mesh=mesh,
        out_type=jax.ShapeDtypeStruct((V * D,), jnp.float32),
        scratch_types=[
            pltpu.VMEM((16, 8, 128), jnp.float32),
            pltpu.VMEM((16384,), jnp.float32),
            pltpu.SemaphoreType.DMA,
            pltpu.SemaphoreType.DMA,
        ],
        compiler_params=pltpu.CompilerParams(
            use_tc_tiling_on_sc=True, needs_layout_passes=False,
            disable_bounds_checks=True),
    )
    def detile_kernel(wT_hbm, tail_hbm, wlin_hbm, tbuf, obuf, sem, osem):
        wid = lax.axis_index("s") * nc + lax.axis_index("c")
        lanes = lax.iota(jnp.int32, _LANES)

        def sb_body(i, carry):
            sb = i * 32 + wid

            @pl.when(sb < SB)
            def _():
                for dblk in range(D // 8):
                    for t in range(4):
                        pltpu.async_copy(
                            wT_hbm.at[pl.ds(dblk * 8, 8),
                                      pl.ds(pl.multiple_of(
                                          (sb * 4 + t) * 128, 128), 128)],
                            tbuf.at[dblk * 4 + t], sem)
                for k in range(16):
                    pltpu.make_async_copy(
                        wT_hbm.at[pl.ds(0, 8), pl.ds(0, 128)],
                        tbuf.at[k], sem).wait()

                def jb_body(jb, carry2):
                    t = jb // 8
                    cvec = (jb % 8) * _LANES + lanes
                    jvec = t * 128 + cvec
                    for dg in range(D // 8):
                        vals, poss = [], []
                        for dd in range(8):
                            rot = (dg * 8 + dd + lanes) & (D - 1)
                            vals.append(plsc.load_gather(
                                tbuf, [(rot >> 3) * 4 + t, rot & 7, cvec]))
                            poss.append(jvec * D + rot)
                        for dd in range(8):
                            plsc.store_scatter(obuf, [poss[dd]], vals[dd])
                    return carry2

                lax.fori_loop(0, 32, jb_body, 0)
                pltpu.async_copy(
                    obuf,
                    wlin_hbm.at[pl.ds(pl.multiple_of(sb * 16384, 16384),
                                      16384)], osem)
                pltpu.make_async_copy(
                    obuf, wlin_hbm.at[pl.ds(0, 16384)], osem).wait()
            return carry

        lax.fori_loop(0, n_iter, sb_body, 0)

        @pl.when(wid == 0)
        def _():
            pltpu.sync_copy(tail_hbm, obuf.at[pl.ds(0, (V % 128) * D)])
            pltpu.sync_copy(obuf.at[pl.ds(0, (V % 128) * D)],
                            wlin_hbm.at[pl.ds(VBF * 128 * D, (V % 128) * D)])

    tail = wT.T[V - V % 128:].reshape((V % 128) * D)
    return detile_kernel(wT, tail)

@functools.partial(jax.jit, static_argnums=(3, 4, 5, 6, 7, 8))
def _masked_gather(weight, mask, widx, V, D, N, NW, B, H):
    PIECES = N // _GSUB
    PPW = PIECES // NW
    n_chunks = PPW // _PP
    BB = B // 128
    DBLK = D // 8
    info = plsc.get_sparse_core_info()
    nc = info.num_cores

    mesh = plsc.VectorSubcoreMesh(core_axis_name="c", subcore_axis_name="s")

    @functools.partial(
        pl.kernel,
        mesh=mesh,
        out_type=jax.ShapeDtypeStruct((N * D,), jnp.float32),
        scratch_types=[
            pltpu.VMEM((2, _CH), jnp.int32),
            pltpu.VMEM((2, _CH), jnp.float32),
            pltpu.VMEM((2, _CH, D), jnp.float32),
            pltpu.VMEM((2, _PP, DBLK * 8 * _GSUB), jnp.float32),
            pltpu.SemaphoreType.DMA,
            pltpu.SemaphoreType.DMA,
            pltpu.SemaphoreType.DMA,
            pltpu.SemaphoreType.DMA,
        ],
        compiler_params=pltpu.CompilerParams(
            use_tc_tiling_on_sc=False, needs_layout_passes=False,
            disable_bounds_checks=True),
    )
    def gather_kernel(weight_hbm, mask_hbm, widx_hbm, out_hbm,
                      idx_v, mval_v, rows_v, obuf,
                      gsem0, gsem1, osem0, osem1):
        wid = lax.axis_index("s") * nc + lax.axis_index("c")
        gsem = (gsem0, gsem1)
        osem = (osem0, osem1)

        def fire_gathers(c, s):
            start = pl.multiple_of((wid * PPW + c * _PP) * _GSUB, _CH)
            pltpu.sync_copy(widx_hbm.at[pl.ds(start, _CH)], idx_v.at[s])
            for j in range(_PP):
                sub = idx_v.at[s, pl.ds(j * _GSUB, _GSUB)]
                pltpu.async_copy(
                    weight_hbm.at[sub],
                    rows_v.at[s, pl.ds(j * _GSUB, _GSUB)], gsem[s])
                pltpu.async_copy(
                    mask_hbm.at[sub],
                    mval_v.at[s, pl.ds(j * _GSUB, _GSUB)], gsem[s])

        def drain_gathers(s):
            pltpu.make_async_copy(
                weight_hbm.at[pl.ds(0, _CH), :], rows_v.at[s], gsem[s]).wait()
            pltpu.make_async_copy(
                mask_hbm.at[pl.ds(0, _CH)], mval_v.at[s], gsem[s]).wait()

        def fire_outs(c, s):
            gp0 = wid * PPW + c * _PP
            for p in range(_PP):
                gp = gp0 + p
                hblk = gp // (BB * 8)
                rem = gp % (BB * 8)
                bblk = rem // 8
                hsub = rem % 8
                h = hblk * 8 + hsub
                base = h * (D * B) + bblk * 1024
                for dblk in range(DBLK):
                    off = pl.multiple_of(base + dblk * (8 * B), 1024)
                    pltpu.async_copy(
                        obuf.at[s, p, pl.ds(dblk * 1024, 1024)],
                        out_hbm.at[pl.ds(off, 1024)], osem[s])

        def drain_outs(s):
            for p in range(_PP):
                pltpu.make_async_copy(
                    obuf.at[s, p], out_hbm.at[pl.ds(0, DBLK * 8 * _GSUB)],
                    osem[s]).wait()

        def compute(s):
            def piece_body(p, carry2):
                lanes = lax.iota(jnp.int32, _LANES)
                for dg in range(D // 8):
                    rots = [(dg * 8 + dd + lanes) & (D - 1) for dd in range(8)]
                    opos = [((r >> 3) << 10) + ((r & 7) << 7) for r in rots]
                    for jb in range(8):
                        rbase = p * _GSUB + jb * _LANES
                        rowvec = rbase + lanes
                        mvec = mval_v[s, pl.ds(rbase, _LANES)]
                        obase = jb * _LANES + lanes
                        vals = [plsc.load_gather(rows_v.at[s], [rowvec, rots[dd]])
                                for dd in range(8)]
                        for dd in range(8):
                            plsc.store_scatter(
                                obuf.at[s, p], [opos[dd] + obase],
                                vals[dd] * mvec)
                return carry2

            lax.fori_loop(0, _PP, piece_body, 0)

        fire_gathers(0, 0)

        def pair_body(cp, carry):
            for s in range(2):
                c = cp * 2 + s

                @pl.when(c + 1 < n_chunks)
                def _():
                    fire_gathers(c + 1, 1 - s)

                drain_gathers(s)

                @pl.when(c >= 2)
                def _():
                    drain_outs(s)

                compute(s)
                fire_outs(c, s)
            return carry

        lax.fori_loop(0, n_chunks // 2, pair_body, 0)
        drain_outs(0)
        drain_outs(1)

    return gather_kernel(weight, mask, widx)


def kernel(weight, words, p):
    V, D = weight.shape
    B, H = words.shape
    N = B * H
    mask = jax.random.bernoulli(
        jax.random.key(42), p, (V,)).astype(jnp.float32) / p
    info = plsc.get_sparse_core_info()
    NW = info.num_cores * info.num_subcores
    widx = (words.astype(jnp.int32)
            .reshape(B // 128, 128, H // 8, 8)
            .transpose(2, 0, 3, 1)
            .reshape(N))
    wlin = _detile_weight(weight.T, V, D).reshape(V, D)
    out1d = _masked_gather(wlin, mask, widx, V, D, N, NW, B, H)
    a = out1d.reshape(H, D // 8, B // 128, 8, 128)
    return a.transpose(2, 4, 0, 1, 3).reshape(B, H, D)
